# Initial kernel scaffold; baseline (speedup 1.0000x reference)
import jax
import jax.numpy as jnp
from jax.experimental import pallas as pl


def kernel(word, pos1, pos2, word_table, pos1_table, pos2_table):
    raise NotImplementedError("write your pallas kernel here")


# speedup vs baseline: 3.5275x; 3.5275x over previous
import functools

import jax
import jax.numpy as jnp
from jax import lax
from jax.experimental import pallas as pl
from jax.experimental.pallas import tpu as pltpu
from jax.experimental.pallas import tpu_sc as plsc

BATCH = 1024
SEQ = 200
WORD_DIM = 128
POS_DIM = 5
NPOS = 400
OUT_DIM = WORD_DIM + 2 * POS_DIM
TOK = BATCH * SEQ

_NW = 32
_PER_W = TOK // _NW
_C = 320
_STEPS = _PER_W // _C
_L = 16


def _make_kernel():
    mesh = plsc.VectorSubcoreMesh(core_axis_name="c", subcore_axis_name="s")

    @functools.partial(
        pl.kernel,
        mesh=mesh,
        compiler_params=pltpu.CompilerParams(
            needs_layout_passes=False, use_tc_tiling_on_sc=False),
        out_type=jax.ShapeDtypeStruct((TOK, OUT_DIM), jnp.float32),
        scratch_types=[
            pltpu.VMEM((_C,), jnp.int32),
            pltpu.VMEM((_C,), jnp.int32),
            pltpu.VMEM((_C,), jnp.int32),
            pltpu.VMEM((NPOS, POS_DIM), jnp.float32),
            pltpu.VMEM((NPOS, POS_DIM), jnp.float32),
            pltpu.VMEM((_C, WORD_DIM), jnp.float32),
            pltpu.VMEM((_C, OUT_DIM), jnp.float32),
            pltpu.SemaphoreType.DMA,
        ],
    )
    def k(word_hbm, p1_hbm, p2_hbm, wt_hbm, p1t_hbm, p2t_hbm, out_hbm,
          widx, p1idx, p2idx, p1t_v, p2t_v, wbuf, obuf, semw):
        wid = lax.axis_index("s") * 2 + lax.axis_index("c")
        pltpu.sync_copy(p1t_hbm, p1t_v)
        pltpu.sync_copy(p2t_hbm, p2t_v)
        lane = lax.iota(jnp.int32, _L)

        def step(i, carry):
            base = wid * _PER_W + i * _C
            pltpu.sync_copy(word_hbm.at[pl.ds(base, _C)], widx)
            pltpu.sync_copy(p1_hbm.at[pl.ds(base, _C)], p1idx)
            pltpu.sync_copy(p2_hbm.at[pl.ds(base, _C)], p2idx)
            cw = pltpu.async_copy(wt_hbm.at[widx], wbuf, semw)

            def pos_group(g, carry2):
                tok = g * _L + lane
                i1 = p1idx[pl.ds(g * _L, _L)]
                i2 = p2idx[pl.ds(g * _L, _L)]
                for dd in range(POS_DIM):
                    dcol = jnp.full((_L,), dd, jnp.int32)
                    v1 = plsc.load_gather(p1t_v, [i1, dcol])
                    plsc.store_scatter(
                        obuf, [tok, jnp.full((_L,), WORD_DIM + dd, jnp.int32)], v1)
                    v2 = plsc.load_gather(p2t_v, [i2, dcol])
                    plsc.store_scatter(
                        obuf, [tok, jnp.full((_L,), WORD_DIM + POS_DIM + dd, jnp.int32)], v2)
                return carry2

            lax.fori_loop(0, _C // _L, pos_group, 0)
            cw.wait()

            def word_copy(t, carry2):
                for c in range(WORD_DIM // _L):
                    obuf[t, pl.ds(c * _L, _L)] = wbuf[t, pl.ds(c * _L, _L)]
                return carry2

            lax.fori_loop(0, _C, word_copy, 0)
            pltpu.sync_copy(obuf, out_hbm.at[pl.ds(base, _C), :])
            return carry

        lax.fori_loop(0, _STEPS, step, 0)

    return k


_k = _make_kernel()


def kernel(word, pos1, pos2, word_table, pos1_table, pos2_table):
    p1t = pos1_table.at[0].set(0.0)
    p2t = pos2_table.at[0].set(0.0)
    out = _k(word.reshape(TOK).astype(jnp.int32),
             pos1.reshape(TOK).astype(jnp.int32),
             pos2.reshape(TOK).astype(jnp.int32),
             word_table, p1t, p2t)
    return out.reshape(BATCH, SEQ, OUT_DIM)
